# Initial kernel scaffold; baseline (speedup 1.0000x reference)
import jax
import jax.numpy as jnp
from jax.experimental import pallas as pl


def kernel(input_, weight):
    raise NotImplementedError("write your pallas kernel here")


# speedup vs baseline: 1.8754x; 1.8754x over previous
import functools

import jax
import jax.numpy as jnp
from jax import lax
from jax.experimental import pallas as pl
from jax.experimental.pallas import tpu as pltpu
from jax.experimental.pallas import tpu_sc as plsc

EMBED_DIM = 64

NC = 2
NS = 16
NW = NC * NS

G = 128
NBUF = 8


@functools.lru_cache(maxsize=None)
def _make_gather(B):
    b_per_w = B // NW
    ngrp = b_per_w // G
    nouter = ngrp // NBUF
    mesh = plsc.VectorSubcoreMesh(core_axis_name="c", subcore_axis_name="s")

    @functools.partial(
        pl.kernel,
        mesh=mesh,
        out_type=jax.ShapeDtypeStruct((B, EMBED_DIM), jnp.float32),
        compiler_params=pltpu.CompilerParams(use_tc_tiling_on_sc=False),
        scratch_types=(
            [
                pltpu.VMEM((ngrp, G), jnp.int32),
                pltpu.VMEM((NBUF, G, EMBED_DIM), jnp.float32),
            ]
            + [pltpu.SemaphoreType.DMA] * NBUF
        ),
    )
    def gather_kernel(idx_hbm, w_hbm, out_hbm, idx_v, rows_v, *sems):
        wid = lax.axis_index("s") * NC + lax.axis_index("c")
        base = wid * b_per_w

        pltpu.sync_copy(idx_hbm.at[wid], idx_v)

        def start(g, b):
            pltpu.make_async_copy(
                w_hbm.at[idx_v.at[g]], rows_v.at[b], sems[b]
            ).start()

        def wait(g, b):
            pltpu.make_async_copy(
                w_hbm.at[idx_v.at[g]], rows_v.at[b], sems[b]
            ).wait()

        def drain(g, b):
            wait(g, b)
            pltpu.sync_copy(rows_v.at[b], out_hbm.at[pl.ds(base + g * G, G)])

        for b in range(NBUF):
            start(b, b)

        def outer(o, carry):
            for b in range(NBUF):
                g = o * NBUF + b
                drain(g, b)
                start(g + NBUF, b)
            return carry

        lax.fori_loop(0, nouter - 1, outer, 0)

        for b in range(NBUF):
            drain((nouter - 1) * NBUF + b, b)

    return gather_kernel


def kernel(input_, weight):
    n, s = input_.shape
    B = n * s
    idx = input_.reshape(NW, B // (NW * G), G).astype(jnp.int32)
    out = _make_gather(B)(idx, weight)
    return out.reshape(n, s, EMBED_DIM)
